# initial kernel scaffold (unmeasured)
import jax
import jax.numpy as jnp
from jax import lax
from jax.experimental import pallas as pl
from jax.experimental.pallas import tpu as pltpu

N_DEV = 4


def kernel(A, B):
    m, k = A.shape
    _, n = B.shape

    def body(a_ref, b_ref, out_ref, comm_ref, send_sems, recv_sems):
        my = lax.axis_index("i")
        left = lax.rem(my + (N_DEV - 1), N_DEV)
        right = lax.rem(my + 1, N_DEV)

        barrier_sem = pltpu.get_barrier_semaphore()
        for nbr in (left, right):
            pl.semaphore_signal(
                barrier_sem, inc=1,
                device_id=(nbr,), device_id_type=pl.DeviceIdType.MESH,
            )
        pl.semaphore_wait(barrier_sem, 2)

        a = a_ref[...].astype(jnp.bfloat16)
        b = b_ref[...].astype(jnp.bfloat16)
        partial = jnp.dot(a, b, preferred_element_type=jnp.float32)
        out_ref[...] = partial
        comm_ref[0, :, :] = partial.astype(jnp.bfloat16)

        for h in range(N_DEV - 1):
            rdma = pltpu.make_async_remote_copy(
                src_ref=comm_ref.at[h],
                dst_ref=comm_ref.at[h + 1],
                send_sem=send_sems.at[h],
                recv_sem=recv_sems.at[h],
                device_id=(right,),
                device_id_type=pl.DeviceIdType.MESH,
            )
            rdma.start()
            rdma.wait()
            out_ref[...] += comm_ref[h + 1, :, :].astype(jnp.float32)

        z = out_ref[...]
        out_ref[...] = z / (1.0 + jnp.exp(-z))

    return pl.pallas_call(
        body,
        out_shape=jax.ShapeDtypeStruct((m, n), jnp.float32),
        in_specs=[
            pl.BlockSpec(memory_space=pltpu.VMEM),
            pl.BlockSpec(memory_space=pltpu.VMEM),
        ],
        out_specs=pl.BlockSpec(memory_space=pltpu.VMEM),
        scratch_shapes=[
            pltpu.VMEM((N_DEV, m, n), jnp.bfloat16),
            pltpu.SemaphoreType.DMA((N_DEV - 1,)),
            pltpu.SemaphoreType.DMA((N_DEV - 1,)),
        ],
        compiler_params=pltpu.CompilerParams(collective_id=0),
    )(A, B)


# baseline (device time: 175702 ns/iter reference)
import jax
import jax.numpy as jnp
from jax import lax
from jax.experimental import pallas as pl
from jax.experimental.pallas import tpu as pltpu

N_DEV = 4


def kernel(A, B):
    m, k = A.shape
    _, n = B.shape
    mc = m // N_DEV

    def body(a_ref, b_ref, out_ref, pbuf, rs_recv, bbf,
             send_sems, recv_sems):
        my = lax.axis_index("i")
        left = lax.rem(my + (N_DEV - 1), N_DEV)
        right = lax.rem(my + 1, N_DEV)

        barrier_sem = pltpu.get_barrier_semaphore()
        for nbr in (left, right):
            pl.semaphore_signal(
                barrier_sem, inc=1,
                device_id=(nbr,), device_id_type=pl.DeviceIdType.MESH,
            )
        pl.semaphore_wait(barrier_sem, 2)

        bbf[...] = b_ref[...].astype(jnp.bfloat16)
        for c in range(N_DEV):
            ac = a_ref[pl.ds(c * mc, mc), :].astype(jnp.bfloat16)
            pc = jnp.dot(ac, bbf[...], preferred_element_type=jnp.float32)
            pbuf[pl.ds(c * mc, mc), :] = pc.astype(jnp.bfloat16)

        for s in range(N_DEV - 1):
            c = lax.rem(my - s + N_DEV, N_DEV)
            if s > 0:
                acc = (rs_recv[s - 1].astype(jnp.float32)
                       + pbuf[pl.ds(c * mc, mc), :].astype(jnp.float32))
                pbuf[pl.ds(c * mc, mc), :] = acc.astype(jnp.bfloat16)
            rdma = pltpu.make_async_remote_copy(
                src_ref=pbuf.at[pl.ds(c * mc, mc), :],
                dst_ref=rs_recv.at[s],
                send_sem=send_sems.at[s],
                recv_sem=recv_sems.at[s],
                device_id=(right,),
                device_id_type=pl.DeviceIdType.MESH,
            )
            rdma.start()
            rdma.wait()

        own = lax.rem(my + 1, N_DEV)
        z = (rs_recv[N_DEV - 2].astype(jnp.float32)
             + pbuf[pl.ds(own * mc, mc), :].astype(jnp.float32))
        out_ref[pl.ds(own * mc, mc), :] = (z * jax.nn.sigmoid(z)).astype(
            jnp.bfloat16)

        for s in range(N_DEV - 1):
            c = lax.rem(my + 1 - s + N_DEV, N_DEV)
            rdma = pltpu.make_async_remote_copy(
                src_ref=out_ref.at[pl.ds(c * mc, mc), :],
                dst_ref=out_ref.at[pl.ds(c * mc, mc), :],
                send_sem=send_sems.at[N_DEV - 1 + s],
                recv_sem=recv_sems.at[N_DEV - 1 + s],
                device_id=(right,),
                device_id_type=pl.DeviceIdType.MESH,
            )
            rdma.start()
            rdma.wait()

    return pl.pallas_call(
        body,
        out_shape=jax.ShapeDtypeStruct((m, n), jnp.bfloat16),
        in_specs=[
            pl.BlockSpec(memory_space=pltpu.VMEM),
            pl.BlockSpec(memory_space=pltpu.VMEM),
        ],
        out_specs=pl.BlockSpec(memory_space=pltpu.VMEM),
        scratch_shapes=[
            pltpu.VMEM((m, n), jnp.bfloat16),
            pltpu.VMEM((N_DEV - 1, mc, n), jnp.bfloat16),
            pltpu.VMEM((k, n), jnp.bfloat16),
            pltpu.SemaphoreType.DMA((2 * (N_DEV - 1),)),
            pltpu.SemaphoreType.DMA((2 * (N_DEV - 1),)),
        ],
        compiler_params=pltpu.CompilerParams(
            collective_id=0,
            vmem_limit_bytes=128 * 1024 * 1024,
        ),
    )(A, B)


# device time: 100945 ns/iter; 1.7406x vs baseline; 1.7406x over previous
import jax
import jax.numpy as jnp
from jax import lax
from jax.experimental import pallas as pl
from jax.experimental.pallas import tpu as pltpu

N_DEV = 4


def kernel(A, B):
    m, k = A.shape
    _, n = B.shape
    mc = m // N_DEV
    nh = n // 2

    def kernel_body(a_ref, b_ref, out_ref, pbuf, rs_cw, rs_ccw, bbf,
                    s_cw, r_cw, s_ccw, r_ccw,
                    ag_s_cw, ag_r_cw, ag_s_ccw, ag_r_ccw):
        my = lax.axis_index("i")
        left = lax.rem(my + (N_DEV - 1), N_DEV)
        right = lax.rem(my + 1, N_DEV)

        barrier_sem = pltpu.get_barrier_semaphore()
        for nbr in (left, right):
            pl.semaphore_signal(
                barrier_sem, inc=1,
                device_id=(nbr,), device_id_type=pl.DeviceIdType.MESH,
            )
        pl.semaphore_wait(barrier_sem, 2)

        bbf[...] = b_ref[...].astype(jnp.bfloat16)

        def compute_chunk(c):
            ac = a_ref[pl.ds(c * mc, mc), :].astype(jnp.bfloat16)
            pc = jnp.dot(ac, bbf[...], preferred_element_type=jnp.float32)
            pbuf[pl.ds(c * mc, mc), :] = pc.astype(jnp.bfloat16)

        def cw_send(s, c):
            d = pltpu.make_async_remote_copy(
                src_ref=pbuf.at[pl.ds(c * mc, mc), pl.ds(0, nh)],
                dst_ref=rs_cw.at[s],
                send_sem=s_cw.at[s], recv_sem=r_cw.at[s],
                device_id=(right,), device_id_type=pl.DeviceIdType.MESH,
            )
            d.start()
            return d

        def ccw_send(s, c):
            d = pltpu.make_async_remote_copy(
                src_ref=pbuf.at[pl.ds(c * mc, mc), pl.ds(nh, nh)],
                dst_ref=rs_ccw.at[s],
                send_sem=s_ccw.at[s], recv_sem=r_ccw.at[s],
                device_id=(left,), device_id_type=pl.DeviceIdType.MESH,
            )
            d.start()
            return d

        pending = []

        compute_chunk(my)
        cw_d = [cw_send(0, my)]
        ccw_d = [ccw_send(0, my)]

        compute_chunk(lax.rem(my + 1, N_DEV))
        compute_chunk(lax.rem(my + 3, N_DEV))
        compute_chunk(lax.rem(my + 2, N_DEV))

        for s in (1, 2):
            c_cw = lax.rem(my - s + N_DEV, N_DEV)
            cw_d[s - 1].wait_recv()
            acc = (rs_cw[s - 1].astype(jnp.float32)
                   + pbuf[pl.ds(c_cw * mc, mc), pl.ds(0, nh)].astype(
                       jnp.float32))
            pbuf[pl.ds(c_cw * mc, mc), pl.ds(0, nh)] = acc.astype(
                jnp.bfloat16)
            cw_d.append(cw_send(s, c_cw))

            c_ccw = lax.rem(my + s, N_DEV)
            ccw_d[s - 1].wait_recv()
            acc = (rs_ccw[s - 1].astype(jnp.float32)
                   + pbuf[pl.ds(c_ccw * mc, mc), pl.ds(nh, nh)].astype(
                       jnp.float32))
            pbuf[pl.ds(c_ccw * mc, mc), pl.ds(nh, nh)] = acc.astype(
                jnp.bfloat16)
            ccw_d.append(ccw_send(s, c_ccw))
        pending.extend(cw_d)
        pending.extend(ccw_d)

        o_cw = lax.rem(my + 1, N_DEV)
        cw_d[2].wait_recv()
        z = (rs_cw[2].astype(jnp.float32)
             + pbuf[pl.ds(o_cw * mc, mc), pl.ds(0, nh)].astype(jnp.float32))
        out_ref[pl.ds(o_cw * mc, mc), pl.ds(0, nh)] = (
            z * jax.nn.sigmoid(z)).astype(jnp.bfloat16)

        o_ccw = lax.rem(my + 3, N_DEV)
        ccw_d[2].wait_recv()
        z = (rs_ccw[2].astype(jnp.float32)
             + pbuf[pl.ds(o_ccw * mc, mc), pl.ds(nh, nh)].astype(
                 jnp.float32))
        out_ref[pl.ds(o_ccw * mc, mc), pl.ds(nh, nh)] = (
            z * jax.nn.sigmoid(z)).astype(jnp.bfloat16)

        ag_cw_d = []
        ag_ccw_d = []
        for s in range(N_DEV - 1):
            g_cw = lax.rem(my + 1 - s + N_DEV, N_DEV)
            if s > 0:
                ag_cw_d[s - 1].wait_recv()
            d = pltpu.make_async_remote_copy(
                src_ref=out_ref.at[pl.ds(g_cw * mc, mc), pl.ds(0, nh)],
                dst_ref=out_ref.at[pl.ds(g_cw * mc, mc), pl.ds(0, nh)],
                send_sem=ag_s_cw.at[s], recv_sem=ag_r_cw.at[s],
                device_id=(right,), device_id_type=pl.DeviceIdType.MESH,
            )
            d.start()
            ag_cw_d.append(d)

            g_ccw = lax.rem(my + 3 + s, N_DEV)
            if s > 0:
                ag_ccw_d[s - 1].wait_recv()
            d = pltpu.make_async_remote_copy(
                src_ref=out_ref.at[pl.ds(g_ccw * mc, mc), pl.ds(nh, nh)],
                dst_ref=out_ref.at[pl.ds(g_ccw * mc, mc), pl.ds(nh, nh)],
                send_sem=ag_s_ccw.at[s], recv_sem=ag_r_ccw.at[s],
                device_id=(left,), device_id_type=pl.DeviceIdType.MESH,
            )
            d.start()
            ag_ccw_d.append(d)
        ag_cw_d[2].wait_recv()
        ag_ccw_d[2].wait_recv()
        pending.extend(ag_cw_d)
        pending.extend(ag_ccw_d)

        for d in pending:
            d.wait_send()

    return pl.pallas_call(
        kernel_body,
        out_shape=jax.ShapeDtypeStruct((m, n), jnp.bfloat16),
        in_specs=[
            pl.BlockSpec(memory_space=pltpu.VMEM),
            pl.BlockSpec(memory_space=pltpu.VMEM),
        ],
        out_specs=pl.BlockSpec(memory_space=pltpu.VMEM),
        scratch_shapes=[
            pltpu.VMEM((m, n), jnp.bfloat16),
            pltpu.VMEM((N_DEV - 1, mc, nh), jnp.bfloat16),
            pltpu.VMEM((N_DEV - 1, mc, nh), jnp.bfloat16),
            pltpu.VMEM((k, n), jnp.bfloat16),
            pltpu.SemaphoreType.DMA((N_DEV - 1,)),
            pltpu.SemaphoreType.DMA((N_DEV - 1,)),
            pltpu.SemaphoreType.DMA((N_DEV - 1,)),
            pltpu.SemaphoreType.DMA((N_DEV - 1,)),
            pltpu.SemaphoreType.DMA((N_DEV - 1,)),
            pltpu.SemaphoreType.DMA((N_DEV - 1,)),
            pltpu.SemaphoreType.DMA((N_DEV - 1,)),
            pltpu.SemaphoreType.DMA((N_DEV - 1,)),
        ],
        compiler_params=pltpu.CompilerParams(
            collective_id=0,
            vmem_limit_bytes=128 * 1024 * 1024,
        ),
    )(A, B)


# device time: 93827 ns/iter; 1.8726x vs baseline; 1.0759x over previous
import jax
import jax.numpy as jnp
from jax import lax
from jax.experimental import pallas as pl
from jax.experimental.pallas import tpu as pltpu

N_DEV = 4
N_SUB = 2


def kernel(A, B):
    m, k = A.shape
    _, n = B.shape
    mc = m // N_DEV
    ms = mc // N_SUB
    nh = n // 2

    def kernel_body(a_ref, b_ref, out_ref, pbuf, rs_cw, rs_ccw, bbf,
                    s_cw, r_cw, s_ccw, r_ccw,
                    ag_s_cw, ag_r_cw, ag_s_ccw, ag_r_ccw):
        my = lax.axis_index("i")
        left = lax.rem(my + (N_DEV - 1), N_DEV)
        right = lax.rem(my + 1, N_DEV)

        barrier_sem = pltpu.get_barrier_semaphore()
        for nbr in (left, right):
            pl.semaphore_signal(
                barrier_sem, inc=1,
                device_id=(nbr,), device_id_type=pl.DeviceIdType.MESH,
            )
        pl.semaphore_wait(barrier_sem, 2)

        bbf[...] = b_ref[...].astype(jnp.bfloat16)

        def compute_half(c, h):
            ac = a_ref[pl.ds(c * mc, mc), :].astype(jnp.bfloat16)
            pc = jnp.dot(ac, bbf[:, pl.ds(h * nh, nh)],
                         preferred_element_type=jnp.float32)
            pbuf[pl.ds(c * mc, mc), pl.ds(h * nh, nh)] = pc.astype(
                jnp.bfloat16)

        def rs_send(dirn, s, j, c):
            col0 = 0 if dirn == 0 else nh
            buf = rs_cw if dirn == 0 else rs_ccw
            ssem = s_cw if dirn == 0 else s_ccw
            rsem = r_cw if dirn == 0 else r_ccw
            tgt = right if dirn == 0 else left
            d = pltpu.make_async_remote_copy(
                src_ref=pbuf.at[pl.ds(c * mc + j * ms, ms),
                                pl.ds(col0, nh)],
                dst_ref=buf.at[s, pl.ds(j * ms, ms), :],
                send_sem=ssem.at[s * N_SUB + j],
                recv_sem=rsem.at[s * N_SUB + j],
                device_id=(tgt,), device_id_type=pl.DeviceIdType.MESH,
            )
            d.start()
            return d

        def ag_send(dirn, s, j, c):
            col0 = 0 if dirn == 0 else nh
            ssem = ag_s_cw if dirn == 0 else ag_s_ccw
            rsem = ag_r_cw if dirn == 0 else ag_r_ccw
            tgt = right if dirn == 0 else left
            sl = (pl.ds(c * mc + j * ms, ms), pl.ds(col0, nh))
            d = pltpu.make_async_remote_copy(
                src_ref=out_ref.at[sl],
                dst_ref=out_ref.at[sl],
                send_sem=ssem.at[s * N_SUB + j],
                recv_sem=rsem.at[s * N_SUB + j],
                device_id=(tgt,), device_id_type=pl.DeviceIdType.MESH,
            )
            d.start()
            return d

        cw = {}
        ccw = {}
        compute_half(my, 0)
        for j in range(N_SUB):
            cw[(0, j)] = rs_send(0, 0, j, my)
        compute_half(my, 1)
        for j in range(N_SUB):
            ccw[(0, j)] = rs_send(1, 0, j, my)

        c_cw1 = lax.rem(my + 3, N_DEV)
        c_ccw1 = lax.rem(my + 1, N_DEV)
        compute_half(c_cw1, 0)
        compute_half(c_ccw1, 1)

        for s in (1, 2):
            c_cw = lax.rem(my - s + N_DEV, N_DEV)
            c_ccw = lax.rem(my + s, N_DEV)
            for j in range(N_SUB):
                cw[(s - 1, j)].wait_recv()
                row = pl.ds(c_cw * mc + j * ms, ms)
                acc = (rs_cw[s - 1, pl.ds(j * ms, ms), :].astype(jnp.float32)
                       + pbuf[row, pl.ds(0, nh)].astype(jnp.float32))
                pbuf[row, pl.ds(0, nh)] = acc.astype(jnp.bfloat16)
                cw[(s, j)] = rs_send(0, s, j, c_cw)

                ccw[(s - 1, j)].wait_recv()
                row = pl.ds(c_ccw * mc + j * ms, ms)
                acc = (rs_ccw[s - 1, pl.ds(j * ms, ms), :].astype(jnp.float32)
                       + pbuf[row, pl.ds(nh, nh)].astype(jnp.float32))
                pbuf[row, pl.ds(nh, nh)] = acc.astype(jnp.bfloat16)
                ccw[(s, j)] = rs_send(1, s, j, c_ccw)
            if s == 1:
                c2 = lax.rem(my + 2, N_DEV)
                compute_half(c2, 0)
                compute_half(c2, 1)

        o_cw = lax.rem(my + 1, N_DEV)
        o_ccw = lax.rem(my + 3, N_DEV)
        compute_half(o_cw, 0)
        compute_half(o_ccw, 1)

        ag_cw = {}
        ag_ccw = {}

        for j in range(N_SUB):
            cw[(2, j)].wait_recv()
        z = (rs_cw[2].astype(jnp.float32)
             + pbuf[pl.ds(o_cw * mc, mc), pl.ds(0, nh)].astype(jnp.float32))
        out_ref[pl.ds(o_cw * mc, mc), pl.ds(0, nh)] = (
            z * jax.nn.sigmoid(z)).astype(jnp.bfloat16)
        for j in range(N_SUB):
            ag_cw[(0, j)] = ag_send(0, 0, j, o_cw)

        for j in range(N_SUB):
            ccw[(2, j)].wait_recv()
        z = (rs_ccw[2].astype(jnp.float32)
             + pbuf[pl.ds(o_ccw * mc, mc), pl.ds(nh, nh)].astype(
                 jnp.float32))
        out_ref[pl.ds(o_ccw * mc, mc), pl.ds(nh, nh)] = (
            z * jax.nn.sigmoid(z)).astype(jnp.bfloat16)
        for j in range(N_SUB):
            ag_ccw[(0, j)] = ag_send(1, 0, j, o_ccw)

        for s in (1, 2):
            g_cw = lax.rem(my + 1 - s + N_DEV, N_DEV)
            g_ccw = lax.rem(my + 3 + s, N_DEV)
            for j in range(N_SUB):
                ag_cw[(s - 1, j)].wait_recv()
                ag_cw[(s, j)] = ag_send(0, s, j, g_cw)
                ag_ccw[(s - 1, j)].wait_recv()
                ag_ccw[(s, j)] = ag_send(1, s, j, g_ccw)
        for j in range(N_SUB):
            ag_cw[(2, j)].wait_recv()
            ag_ccw[(2, j)].wait_recv()

        for d in (list(cw.values()) + list(ccw.values())
                  + list(ag_cw.values()) + list(ag_ccw.values())):
            d.wait_send()

    n_sem = (N_DEV - 1) * N_SUB
    return pl.pallas_call(
        kernel_body,
        out_shape=jax.ShapeDtypeStruct((m, n), jnp.bfloat16),
        in_specs=[
            pl.BlockSpec(memory_space=pltpu.VMEM),
            pl.BlockSpec(memory_space=pltpu.VMEM),
        ],
        out_specs=pl.BlockSpec(memory_space=pltpu.VMEM),
        scratch_shapes=[
            pltpu.VMEM((m, n), jnp.bfloat16),
            pltpu.VMEM((N_DEV - 1, mc, nh), jnp.bfloat16),
            pltpu.VMEM((N_DEV - 1, mc, nh), jnp.bfloat16),
            pltpu.VMEM((k, n), jnp.bfloat16),
            pltpu.SemaphoreType.DMA((n_sem,)),
            pltpu.SemaphoreType.DMA((n_sem,)),
            pltpu.SemaphoreType.DMA((n_sem,)),
            pltpu.SemaphoreType.DMA((n_sem,)),
            pltpu.SemaphoreType.DMA((n_sem,)),
            pltpu.SemaphoreType.DMA((n_sem,)),
            pltpu.SemaphoreType.DMA((n_sem,)),
            pltpu.SemaphoreType.DMA((n_sem,)),
        ],
        compiler_params=pltpu.CompilerParams(
            collective_id=0,
            vmem_limit_bytes=128 * 1024 * 1024,
        ),
    )(A, B)
